# Initial kernel scaffold; baseline (speedup 1.0000x reference)
import jax
import jax.numpy as jnp
from jax.experimental import pallas as pl


def kernel(x, edge_index, W_ih, W_hh, b_ih, b_hh):
    raise NotImplementedError("write your pallas kernel here")


# speedup vs baseline: 4.8475x; 4.8475x over previous
import functools

import jax
import jax.numpy as jnp
from jax import lax
from jax.experimental import pallas as pl
from jax.experimental.pallas import tpu as pltpu
from jax.experimental.pallas import tpu_sc as plsc

NC = 2
NS = 16
NW = NC * NS

C = 80
ZR = 48
RPT = 624


def _sc_aggregate(x, src_r, dst_r):
    n, d = x.shape
    k_chunks = src_r.shape[0] // (NW * C)
    extra = n - NS * RPT

    mesh = plsc.VectorSubcoreMesh(
        core_axis_name="c", subcore_axis_name="s",
        num_cores=NC, num_subcores=NS)

    @functools.partial(
        pl.kernel,
        out_type=(
            jax.ShapeDtypeStruct((NC, n, d), jnp.float32),
            jax.ShapeDtypeStruct((NC, n, d), jnp.float32),
        ),
        mesh=mesh,
        scratch_types=[
            pltpu.VMEM((C,), jnp.int32),
            pltpu.VMEM((C,), jnp.int32),
            pltpu.VMEM((C, d), jnp.float32),
            pltpu.VMEM((ZR, d), jnp.float32),
            pltpu.VMEM((ZR,), jnp.int32),
            pltpu.VMEM((16,), jnp.int32),
            pltpu.VMEM_SHARED((n, d), jnp.float32),
            pltpu.SemaphoreType.DMA,
        ],
    )
    def agg(x_hbm, src_hbm, dst_hbm, out_feat, out_deg,
            sidx, didx, rows, zf, ridx, tidx, acc_sh, sem):
        c = lax.axis_index("c")
        s = lax.axis_index("s")
        wid = s * NC + c

        zero16 = jnp.zeros((16,), jnp.float32)
        one16 = jnp.ones((16,), jnp.float32)
        iota16 = lax.iota(jnp.int32, 16)
        base = s * RPT
        ebase = wid * (k_chunks * C)

        def fill_zf(r, carry):
            for j in range(d // 16):
                zf[r, pl.ds(j * 16, 16)] = zero16
            return carry

        def zero_acc():
            def zero_batch(q, carry):
                for j in range(ZR // 16):
                    ridx[pl.ds(j * 16, 16)] = base + q * ZR + j * 16 + iota16
                pltpu.sync_copy(zf, acc_sh.at[ridx])
                return carry

            lax.fori_loop(0, RPT // ZR, zero_batch, 0)
            tidx[...] = NS * RPT + iota16
            pltpu.sync_copy(zf.at[pl.ds(0, 16), :], acc_sh.at[tidx])

        def write_out(dst_hbm3):
            def wout(q, carry):
                row = base + q * ZR
                for j in range(ZR // 16):
                    ridx[pl.ds(j * 16, 16)] = row + j * 16 + iota16
                pltpu.async_copy(acc_sh.at[ridx], zf, sem).wait()
                pltpu.sync_copy(zf, dst_hbm3.at[c, pl.ds(row, ZR), :])
                return carry

            lax.fori_loop(0, RPT // ZR, wout, 0)
            pltpu.async_copy(acc_sh.at[tidx], zf.at[pl.ds(0, 16), :],
                             sem).wait()
            pltpu.sync_copy(zf.at[pl.ds(0, 16), :],
                            dst_hbm3.at[c, pl.ds(NS * RPT, 16), :])

        lax.fori_loop(0, ZR, fill_zf, 0)
        zero_acc()
        plsc.subcore_barrier()

        def feat_body(kk, carry):
            off = ebase + kk * C
            pltpu.sync_copy(src_hbm.at[pl.ds(off, C)], sidx)
            pltpu.sync_copy(dst_hbm.at[pl.ds(off, C)], didx)
            pltpu.async_copy(x_hbm.at[sidx], rows, sem).wait()
            pltpu.sync_copy(rows, acc_sh.at[didx], add=True)
            return carry

        lax.fori_loop(0, k_chunks, feat_body, 0)
        plsc.subcore_barrier()
        write_out(out_feat)
        plsc.subcore_barrier()

        lax.fori_loop(0, ZR, fill_zf, 0)
        zero_acc()

        def fill_ones(r, carry):
            for j in range(d // 16):
                rows[r, pl.ds(j * 16, 16)] = one16
            return carry

        lax.fori_loop(0, C, fill_ones, 0)
        plsc.subcore_barrier()

        def deg_body(kk, carry):
            off = ebase + kk * C
            pltpu.sync_copy(dst_hbm.at[pl.ds(off, C)], didx)
            pltpu.sync_copy(rows, acc_sh.at[didx], add=True)
            return carry

        lax.fori_loop(0, k_chunks, deg_body, 0)
        plsc.subcore_barrier()
        write_out(out_deg)

    return agg(x, src_r, dst_r)


def _tc_lstm(x, wih_t, whh_t, b, acc, degp):
    n, d = x.shape
    h = d // 2
    rows = 1000
    grid = n // rows

    def body(x_ref, acc_ref, deg_ref, wih_ref, whh_ref, b_ref, out_ref):
        deg = deg_ref[0, :, 0:1] + deg_ref[1, :, 0:1]
        ft = (acc_ref[0] + acc_ref[1]) / jnp.maximum(deg, 1.0)
        g_t = ft[:, :h]
        r_t = ft[:, h:]
        gates = jnp.dot(x_ref[...], wih_ref[...],
                        preferred_element_type=jnp.float32)
        gates += jnp.dot(g_t, whh_ref[...],
                         preferred_element_type=jnp.float32)
        gates += b_ref[...]
        i_g = jax.nn.sigmoid(gates[:, :h])
        f_g = jax.nn.sigmoid(gates[:, h:2 * h])
        g_g = jnp.tanh(gates[:, 2 * h:3 * h])
        o_g = jax.nn.sigmoid(gates[:, 3 * h:])
        c2 = f_g * r_t + i_g * g_g
        h2 = o_g * jnp.tanh(c2)
        out_ref[...] = jnp.concatenate([h2, c2], axis=1)

    return pl.pallas_call(
        body,
        grid=(grid,),
        in_specs=[
            pl.BlockSpec((rows, d), lambda i: (i, 0)),
            pl.BlockSpec((NC, rows, d), lambda i: (0, i, 0)),
            pl.BlockSpec((NC, rows, d), lambda i: (0, i, 0)),
            pl.BlockSpec((d, 4 * h), lambda i: (0, 0)),
            pl.BlockSpec((h, 4 * h), lambda i: (0, 0)),
            pl.BlockSpec((1, 4 * h), lambda i: (0, 0)),
        ],
        out_specs=pl.BlockSpec((rows, d), lambda i: (i, 0)),
        out_shape=jax.ShapeDtypeStruct((n, d), jnp.float32),
    )(x, acc, degp, wih_t, whh_t, b)


def kernel(x, edge_index, W_ih, W_hh, b_ih, b_hh):
    n, d = x.shape
    acc, degp = _sc_aggregate(x, edge_index[0], edge_index[1])
    b = (b_ih + b_hh).reshape(1, 2 * d)
    return _tc_lstm(x, W_ih.T, W_hh.T, b, acc, degp)


# speedup vs baseline: 7.7241x; 1.5934x over previous
import functools

import jax
import jax.numpy as jnp
from jax import lax
from jax.experimental import pallas as pl
from jax.experimental.pallas import tpu as pltpu
from jax.experimental.pallas import tpu_sc as plsc

NC = 2
NS = 16
NW = NC * NS

C = 80
ZR = 48
RPT = 624


def _sc_aggregate(x, src_r, dst_r):
    n, d = x.shape
    k_chunks = src_r.shape[0] // (NW * C)
    extra = n - NS * RPT

    mesh = plsc.VectorSubcoreMesh(
        core_axis_name="c", subcore_axis_name="s",
        num_cores=NC, num_subcores=NS)

    @functools.partial(
        pl.kernel,
        out_type=(
            jax.ShapeDtypeStruct((NC, n, d), jnp.float32),
            jax.ShapeDtypeStruct((NC, n, d), jnp.float32),
        ),
        mesh=mesh,
        scratch_types=[
            pltpu.VMEM((C,), jnp.int32),
            pltpu.VMEM((C,), jnp.int32),
            pltpu.VMEM((C,), jnp.int32),
            pltpu.VMEM((C,), jnp.int32),
            pltpu.VMEM((C, d), jnp.float32),
            pltpu.VMEM((C, d), jnp.float32),
            pltpu.VMEM((ZR, d), jnp.float32),
            pltpu.VMEM((ZR,), jnp.int32),
            pltpu.VMEM((16,), jnp.int32),
            pltpu.VMEM_SHARED((n, d), jnp.float32),
            pltpu.SemaphoreType.DMA,
            pltpu.SemaphoreType.DMA,
        ],
    )
    def agg(x_hbm, src_hbm, dst_hbm, out_feat, out_deg,
            sidx0, didx0, sidx1, didx1, rows0, rows1, zf, ridx, tidx,
            acc_sh, sem, sem1):
        c = lax.axis_index("c")
        s = lax.axis_index("s")
        wid = s * NC + c

        zero16 = jnp.zeros((16,), jnp.float32)
        one16 = jnp.ones((16,), jnp.float32)
        iota16 = lax.iota(jnp.int32, 16)
        base = s * RPT
        ebase = wid * (k_chunks * C)

        def fill_zf(r, carry):
            for j in range(d // 16):
                zf[r, pl.ds(j * 16, 16)] = zero16
            return carry

        def zero_acc():
            def zero_batch(q, carry):
                for j in range(ZR // 16):
                    ridx[pl.ds(j * 16, 16)] = base + q * ZR + j * 16 + iota16
                pltpu.sync_copy(zf, acc_sh.at[ridx])
                return carry

            lax.fori_loop(0, RPT // ZR, zero_batch, 0)
            tidx[...] = NS * RPT + iota16
            pltpu.sync_copy(zf.at[pl.ds(0, 16), :], acc_sh.at[tidx])

        def write_out(dst_hbm3):
            def wout(q, carry):
                row = base + q * ZR
                for j in range(ZR // 16):
                    ridx[pl.ds(j * 16, 16)] = row + j * 16 + iota16
                pltpu.async_copy(acc_sh.at[ridx], zf, sem).wait()
                pltpu.sync_copy(zf, dst_hbm3.at[c, pl.ds(row, ZR), :])
                return carry

            lax.fori_loop(0, RPT // ZR, wout, 0)
            pltpu.async_copy(acc_sh.at[tidx], zf.at[pl.ds(0, 16), :],
                             sem).wait()
            pltpu.sync_copy(zf.at[pl.ds(0, 16), :],
                            dst_hbm3.at[c, pl.ds(NS * RPT, 16), :])

        def load_idx(kk, si, di):
            off = ebase + kk * C
            pltpu.sync_copy(src_hbm.at[pl.ds(off, C)], si)
            pltpu.sync_copy(dst_hbm.at[pl.ds(off, C)], di)

        lax.fori_loop(0, ZR, fill_zf, 0)
        zero_acc()
        plsc.subcore_barrier()

        load_idx(0, sidx0, didx0)
        pltpu.async_copy(x_hbm.at[sidx0], rows0, sem)

        def feat_body(p, carry):
            kb = 2 * p + 1
            load_idx(kb, sidx1, didx1)
            pltpu.async_copy(x_hbm.at[sidx1], rows1, sem1)
            pltpu.make_async_copy(x_hbm.at[sidx0], rows0, sem).wait()
            pltpu.sync_copy(rows0, acc_sh.at[didx0], add=True)
            load_idx(kb + 1, sidx0, didx0)
            pltpu.async_copy(x_hbm.at[sidx0], rows0, sem)
            pltpu.make_async_copy(x_hbm.at[sidx1], rows1, sem1).wait()
            pltpu.sync_copy(rows1, acc_sh.at[didx1], add=True)
            return carry

        lax.fori_loop(0, (k_chunks - 1) // 2, feat_body, 0)
        pltpu.make_async_copy(x_hbm.at[sidx0], rows0, sem).wait()
        pltpu.sync_copy(rows0, acc_sh.at[didx0], add=True)
        plsc.subcore_barrier()
        write_out(out_feat)
        plsc.subcore_barrier()

        lax.fori_loop(0, ZR, fill_zf, 0)
        zero_acc()

        def fill_ones(r, carry):
            for j in range(d // 16):
                rows0[r, pl.ds(j * 16, 16)] = one16
            return carry

        lax.fori_loop(0, C, fill_ones, 0)
        plsc.subcore_barrier()

        pltpu.sync_copy(dst_hbm.at[pl.ds(ebase, C)], didx0)

        def deg_body(p, carry):
            kb = 2 * p + 1
            pltpu.async_copy(
                dst_hbm.at[pl.ds(ebase + kb * C, C)], didx1, sem1)
            pltpu.sync_copy(rows0, acc_sh.at[didx0], add=True)
            pltpu.make_async_copy(
                dst_hbm.at[pl.ds(ebase + kb * C, C)], didx1, sem1).wait()
            pltpu.async_copy(
                dst_hbm.at[pl.ds(ebase + (kb + 1) * C, C)], didx0, sem)
            pltpu.sync_copy(rows0, acc_sh.at[didx1], add=True)
            pltpu.make_async_copy(
                dst_hbm.at[pl.ds(ebase + (kb + 1) * C, C)], didx0, sem).wait()
            return carry

        lax.fori_loop(0, (k_chunks - 1) // 2, deg_body, 0)
        pltpu.sync_copy(rows0, acc_sh.at[didx0], add=True)
        plsc.subcore_barrier()
        write_out(out_deg)

    return agg(x, src_r, dst_r)


def _tc_lstm(x, wih_t, whh_t, b, acc, degp):
    n, d = x.shape
    h = d // 2
    rows = 1000
    grid = n // rows

    def body(x_ref, acc_ref, deg_ref, wih_ref, whh_ref, b_ref, out_ref):
        deg = deg_ref[0, :, 0:1] + deg_ref[1, :, 0:1]
        ft = (acc_ref[0] + acc_ref[1]) / jnp.maximum(deg, 1.0)
        g_t = ft[:, :h]
        r_t = ft[:, h:]
        gates = jnp.dot(x_ref[...], wih_ref[...],
                        preferred_element_type=jnp.float32)
        gates += jnp.dot(g_t, whh_ref[...],
                         preferred_element_type=jnp.float32)
        gates += b_ref[...]
        i_g = jax.nn.sigmoid(gates[:, :h])
        f_g = jax.nn.sigmoid(gates[:, h:2 * h])
        g_g = jnp.tanh(gates[:, 2 * h:3 * h])
        o_g = jax.nn.sigmoid(gates[:, 3 * h:])
        c2 = f_g * r_t + i_g * g_g
        h2 = o_g * jnp.tanh(c2)
        out_ref[...] = jnp.concatenate([h2, c2], axis=1)

    return pl.pallas_call(
        body,
        grid=(grid,),
        in_specs=[
            pl.BlockSpec((rows, d), lambda i: (i, 0)),
            pl.BlockSpec((NC, rows, d), lambda i: (0, i, 0)),
            pl.BlockSpec((NC, rows, d), lambda i: (0, i, 0)),
            pl.BlockSpec((d, 4 * h), lambda i: (0, 0)),
            pl.BlockSpec((h, 4 * h), lambda i: (0, 0)),
            pl.BlockSpec((1, 4 * h), lambda i: (0, 0)),
        ],
        out_specs=pl.BlockSpec((rows, d), lambda i: (i, 0)),
        out_shape=jax.ShapeDtypeStruct((n, d), jnp.float32),
    )(x, acc, degp, wih_t, whh_t, b)


def kernel(x, edge_index, W_ih, W_hh, b_ih, b_hh):
    n, d = x.shape
    acc, degp = _sc_aggregate(x, edge_index[0], edge_index[1])
    b = (b_ih + b_hh).reshape(1, 2 * d)
    return _tc_lstm(x, W_ih.T, W_hh.T, b, acc, degp)


# speedup vs baseline: 8.9349x; 1.1568x over previous
import functools

import jax
import jax.numpy as jnp
from jax import lax
from jax.experimental import pallas as pl
from jax.experimental.pallas import tpu as pltpu
from jax.experimental.pallas import tpu_sc as plsc

NC = 2
NS = 16
NW = NC * NS

C = 80
ZR = 16
DB = 400
RPT = 624


def _sc_aggregate(x, src_r, dst_r):
    n, d = x.shape
    k_chunks = src_r.shape[0] // (NW * C)
    extra = n - NS * RPT

    mesh = plsc.VectorSubcoreMesh(
        core_axis_name="c", subcore_axis_name="s",
        num_cores=NC, num_subcores=NS)

    @functools.partial(
        pl.kernel,
        out_type=(
            jax.ShapeDtypeStruct((NC, n, d), jnp.float32),
            jax.ShapeDtypeStruct((NW, 1, 10240), jnp.float32),
        ),
        mesh=mesh,
        compiler_params=pltpu.CompilerParams(needs_layout_passes=False),
        scratch_types=[
            pltpu.VMEM((C,), jnp.int32),
            pltpu.VMEM((C,), jnp.int32),
            pltpu.VMEM((C,), jnp.int32),
            pltpu.VMEM((C,), jnp.int32),
            pltpu.VMEM((C, d), jnp.float32),
            pltpu.VMEM((C, d), jnp.float32),
            pltpu.VMEM((ZR, d), jnp.float32),
            pltpu.VMEM((ZR,), jnp.int32),
            pltpu.VMEM((16,), jnp.int32),
            pltpu.VMEM((10240,), jnp.float32),
            pltpu.VMEM((DB,), jnp.int32),
            pltpu.VMEM((DB,), jnp.int32),
            pltpu.VMEM_SHARED((n, d), jnp.float32),
            pltpu.SemaphoreType.DMA,
            pltpu.SemaphoreType.DMA,
        ],
    )
    def agg(x_hbm, src_hbm, dst_hbm, out_feat, out_deg,
            sidx0, didx0, sidx1, didx1, rows0, rows1, zf, ridx, tidx,
            degh, dbig0, dbig1, acc_sh, sem, sem1):
        c = lax.axis_index("c")
        s = lax.axis_index("s")
        wid = s * NC + c

        zero16 = jnp.zeros((16,), jnp.float32)
        one16 = jnp.ones((16,), jnp.float32)
        iota16 = lax.iota(jnp.int32, 16)
        base = s * RPT
        ebase = wid * (k_chunks * C)

        def fill_zf(r, carry):
            for j in range(d // 16):
                zf[r, pl.ds(j * 16, 16)] = zero16
            return carry

        def zero_acc():
            def zero_batch(q, carry):
                for j in range(ZR // 16):
                    ridx[pl.ds(j * 16, 16)] = base + q * ZR + j * 16 + iota16
                pltpu.sync_copy(zf, acc_sh.at[ridx])
                return carry

            lax.fori_loop(0, RPT // ZR, zero_batch, 0)
            tidx[...] = NS * RPT + iota16
            pltpu.sync_copy(zf.at[pl.ds(0, 16), :], acc_sh.at[tidx])

        def write_out(dst_hbm3):
            def wout(q, carry):
                row = base + q * ZR
                for j in range(ZR // 16):
                    ridx[pl.ds(j * 16, 16)] = row + j * 16 + iota16
                pltpu.async_copy(acc_sh.at[ridx], zf, sem).wait()
                pltpu.sync_copy(zf, dst_hbm3.at[c, pl.ds(row, ZR), :])
                return carry

            lax.fori_loop(0, RPT // ZR, wout, 0)
            pltpu.async_copy(acc_sh.at[tidx], zf.at[pl.ds(0, 16), :],
                             sem).wait()
            pltpu.sync_copy(zf.at[pl.ds(0, 16), :],
                            dst_hbm3.at[c, pl.ds(NS * RPT, 16), :])

        def load_idx(kk, si, di):
            off = ebase + kk * C
            pltpu.sync_copy(src_hbm.at[pl.ds(off, C)], si)
            pltpu.sync_copy(dst_hbm.at[pl.ds(off, C)], di)

        lax.fori_loop(0, ZR, fill_zf, 0)
        zero_acc()
        plsc.subcore_barrier()

        load_idx(0, sidx0, didx0)
        pltpu.async_copy(x_hbm.at[sidx0], rows0, sem)

        def feat_body(p, carry):
            kb = 2 * p + 1
            load_idx(kb, sidx1, didx1)
            pltpu.async_copy(x_hbm.at[sidx1], rows1, sem1)
            pltpu.make_async_copy(x_hbm.at[sidx0], rows0, sem).wait()
            pltpu.sync_copy(rows0, acc_sh.at[didx0], add=True)
            load_idx(kb + 1, sidx0, didx0)
            pltpu.async_copy(x_hbm.at[sidx0], rows0, sem)
            pltpu.make_async_copy(x_hbm.at[sidx1], rows1, sem1).wait()
            pltpu.sync_copy(rows1, acc_sh.at[didx1], add=True)
            return carry

        lax.fori_loop(0, (k_chunks - 1) // 2, feat_body, 0)
        pltpu.make_async_copy(x_hbm.at[sidx0], rows0, sem).wait()
        pltpu.sync_copy(rows0, acc_sh.at[didx0], add=True)
        plsc.subcore_barrier()
        write_out(out_feat)
        plsc.subcore_barrier()

        def zero_degh(r, carry):
            degh[pl.ds(r * 16, 16)] = zero16
            return carry

        lax.fori_loop(0, 640, zero_degh, 0)

        one16f = jnp.ones((16,), jnp.float32)
        n_blocks = (k_chunks * C) // DB

        def hist(buf):
            for j in range(DB // 16):
                v = buf[pl.ds(j * 16, 16)]
                plsc.addupdate_scatter(degh, [v], one16f)

        pltpu.sync_copy(dst_hbm.at[pl.ds(ebase, DB)], dbig0)

        def hist_body(p, carry):
            b = 2 * p + 1
            pltpu.async_copy(dst_hbm.at[pl.ds(ebase + b * DB, DB)],
                             dbig1, sem1)
            hist(dbig0)
            pltpu.make_async_copy(dst_hbm.at[pl.ds(ebase + b * DB, DB)],
                                  dbig1, sem1).wait()
            pltpu.async_copy(dst_hbm.at[pl.ds(ebase + (b + 1) * DB, DB)],
                             dbig0, sem)
            hist(dbig1)
            pltpu.make_async_copy(
                dst_hbm.at[pl.ds(ebase + (b + 1) * DB, DB)], dbig0,
                sem).wait()
            return carry

        lax.fori_loop(0, (n_blocks - 1) // 2, hist_body, 0)
        hist(dbig0)

        pltpu.sync_copy(degh, out_deg.at[wid, 0])

    return agg(x, src_r, dst_r)


def _tc_lstm(x, wih_t, whh_t, b, acc, degp):
    n, d = x.shape
    h = d // 2
    rows = 1000
    grid = n // rows

    def body(x_ref, acc_ref, deg_ref, wih_ref, whh_ref, b_ref, out_ref):
        ft = (acc_ref[0] + acc_ref[1]) / jnp.maximum(deg_ref[...], 1.0)
        g_t = ft[:, :h]
        r_t = ft[:, h:]
        gates = jnp.dot(x_ref[...], wih_ref[...],
                        preferred_element_type=jnp.float32)
        gates += jnp.dot(g_t, whh_ref[...],
                         preferred_element_type=jnp.float32)
        gates += b_ref[...]
        i_g = jax.nn.sigmoid(gates[:, :h])
        f_g = jax.nn.sigmoid(gates[:, h:2 * h])
        g_g = jnp.tanh(gates[:, 2 * h:3 * h])
        o_g = jax.nn.sigmoid(gates[:, 3 * h:])
        c2 = f_g * r_t + i_g * g_g
        h2 = o_g * jnp.tanh(c2)
        out_ref[...] = jnp.concatenate([h2, c2], axis=1)

    return pl.pallas_call(
        body,
        grid=(grid,),
        in_specs=[
            pl.BlockSpec((rows, d), lambda i: (i, 0)),
            pl.BlockSpec((NC, rows, d), lambda i: (0, i, 0)),
            pl.BlockSpec((rows, 1), lambda i: (i, 0)),
            pl.BlockSpec((d, 4 * h), lambda i: (0, 0)),
            pl.BlockSpec((h, 4 * h), lambda i: (0, 0)),
            pl.BlockSpec((1, 4 * h), lambda i: (0, 0)),
        ],
        out_specs=pl.BlockSpec((rows, d), lambda i: (i, 0)),
        out_shape=jax.ShapeDtypeStruct((n, d), jnp.float32),
    )(x, acc, degp, wih_t, whh_t, b)


def kernel(x, edge_index, W_ih, W_hh, b_ih, b_hh):
    n, d = x.shape
    acc, degpacked = _sc_aggregate(x, edge_index[0], edge_index[1])
    deg = degpacked.reshape(NW, 10240)[:, :n].sum(axis=0)[:, None]
    b = (b_ih + b_hh).reshape(1, 2 * d)
    return _tc_lstm(x, W_ih.T, W_hh.T, b, acc, deg)


# speedup vs baseline: 9.7184x; 1.0877x over previous
import functools

import jax
import jax.numpy as jnp
from jax import lax
from jax.experimental import pallas as pl
from jax.experimental.pallas import tpu as pltpu
from jax.experimental.pallas import tpu_sc as plsc

NC = 2
NS = 16
NW = NC * NS

C = 80
RPT = 624


def _sc_aggregate(x, src_r, dst_r):
    n, d = x.shape
    k_chunks = src_r.shape[0] // (NW * C)
    extra = n - NS * RPT

    mesh = plsc.VectorSubcoreMesh(
        core_axis_name="c", subcore_axis_name="s",
        num_cores=NC, num_subcores=NS)

    @functools.partial(
        pl.kernel,
        out_type=(
            jax.ShapeDtypeStruct((NC, n, d), jnp.float32),
            jax.ShapeDtypeStruct((NW, 1, 10240), jnp.float32),
        ),
        mesh=mesh,
        compiler_params=pltpu.CompilerParams(needs_layout_passes=False),
        scratch_types=[
            pltpu.VMEM((C,), jnp.int32),
            pltpu.VMEM((C,), jnp.int32),
            pltpu.VMEM((C,), jnp.int32),
            pltpu.VMEM((C,), jnp.int32),
            pltpu.VMEM((C, d), jnp.float32),
            pltpu.VMEM((C, d), jnp.float32),
            pltpu.VMEM((10240,), jnp.float32),
            pltpu.VMEM_SHARED((n, d), jnp.float32),
            pltpu.SemaphoreType.DMA,
            pltpu.SemaphoreType.DMA,
        ],
    )
    def agg(x_hbm, src_hbm, dst_hbm, out_feat, out_deg,
            sidx0, didx0, sidx1, didx1, rows0, rows1,
            degh, acc_sh, sem, sem1):
        c = lax.axis_index("c")
        s = lax.axis_index("s")
        wid = s * NC + c

        zero16 = jnp.zeros((16,), jnp.float32)
        one16 = jnp.ones((16,), jnp.float32)
        iota16 = lax.iota(jnp.int32, 16)
        base = s * RPT
        ebase = wid * (k_chunks * C)

        def load_idx(kk, si, di):
            off = ebase + kk * C
            pltpu.sync_copy(src_hbm.at[pl.ds(off, C)], si)
            pltpu.sync_copy(dst_hbm.at[pl.ds(off, C)], di)

        def hist(di):
            for j in range(C // 16):
                v = di[pl.ds(j * 16, 16)]
                plsc.addupdate_scatter(degh, [v], one16)

        def fill_batch_idx(si, q):
            for j in range(C // 16):
                si[pl.ds(j * 16, 16)] = base + q * C + j * 16 + iota16

        def zero_degh(r, carry):
            degh[pl.ds(r * 16, 16)] = zero16
            return carry

        lax.fori_loop(0, 640, zero_degh, 0)

        def zero_rows0(r, carry):
            for j in range(d // 16):
                rows0[r, pl.ds(j * 16, 16)] = zero16
            return carry

        lax.fori_loop(0, C, zero_rows0, 0)
        for q in range(8):
            fill_batch_idx(sidx0, q)
            pltpu.sync_copy(rows0, acc_sh.at[sidx0])
        plsc.subcore_barrier()

        load_idx(0, sidx0, didx0)
        pltpu.async_copy(x_hbm.at[sidx0], rows0, sem)
        hist(didx0)

        def feat_body(p, carry):
            kb = 2 * p + 1
            load_idx(kb, sidx1, didx1)
            pltpu.async_copy(x_hbm.at[sidx1], rows1, sem1)
            hist(didx1)
            pltpu.make_async_copy(x_hbm.at[sidx0], rows0, sem).wait()
            pltpu.sync_copy(rows0, acc_sh.at[didx0], add=True)
            load_idx(kb + 1, sidx0, didx0)
            pltpu.async_copy(x_hbm.at[sidx0], rows0, sem)
            hist(didx0)
            pltpu.make_async_copy(x_hbm.at[sidx1], rows1, sem1).wait()
            pltpu.sync_copy(rows1, acc_sh.at[didx1], add=True)
            return carry

        lax.fori_loop(0, (k_chunks - 1) // 2, feat_body, 0)
        pltpu.make_async_copy(x_hbm.at[sidx0], rows0, sem).wait()
        pltpu.sync_copy(rows0, acc_sh.at[didx0], add=True)
        pltpu.sync_copy(degh, out_deg.at[wid, 0])
        plsc.subcore_barrier()

        fill_batch_idx(sidx0, 0)
        pltpu.async_copy(acc_sh.at[sidx0], rows0, sem)
        for q in range(8):
            buf, sidx, sm = ((rows0, sidx0, sem) if q % 2 == 0
                             else (rows1, sidx1, sem1))
            if q < 7:
                nbuf, nsidx, nsm = ((rows0, sidx0, sem) if q % 2 == 1
                                    else (rows1, sidx1, sem1))
                fill_batch_idx(nsidx, q + 1)
                pltpu.async_copy(acc_sh.at[nsidx], nbuf, nsm)
            pltpu.make_async_copy(acc_sh.at[sidx], buf, sm).wait()
            pltpu.sync_copy(buf, out_feat.at[c, pl.ds(base + q * C, C), :])

    return agg(x, src_r, dst_r)


def _tc_lstm(x, wih_t, whh_t, b, acc, degp):
    n, d = x.shape
    h = d // 2
    rows = 1000
    grid = n // rows

    def body(x_ref, acc_ref, deg_ref, wih_ref, whh_ref, b_ref, out_ref):
        ft = (acc_ref[0] + acc_ref[1]) / jnp.maximum(deg_ref[...], 1.0)
        g_t = ft[:, :h]
        r_t = ft[:, h:]
        gates = jnp.dot(x_ref[...], wih_ref[...],
                        preferred_element_type=jnp.float32)
        gates += jnp.dot(g_t, whh_ref[...],
                         preferred_element_type=jnp.float32)
        gates += b_ref[...]
        i_g = jax.nn.sigmoid(gates[:, :h])
        f_g = jax.nn.sigmoid(gates[:, h:2 * h])
        g_g = jnp.tanh(gates[:, 2 * h:3 * h])
        o_g = jax.nn.sigmoid(gates[:, 3 * h:])
        c2 = f_g * r_t + i_g * g_g
        h2 = o_g * jnp.tanh(c2)
        out_ref[...] = jnp.concatenate([h2, c2], axis=1)

    return pl.pallas_call(
        body,
        grid=(grid,),
        in_specs=[
            pl.BlockSpec((rows, d), lambda i: (i, 0)),
            pl.BlockSpec((NC, rows, d), lambda i: (0, i, 0)),
            pl.BlockSpec((rows, 1), lambda i: (i, 0)),
            pl.BlockSpec((d, 4 * h), lambda i: (0, 0)),
            pl.BlockSpec((h, 4 * h), lambda i: (0, 0)),
            pl.BlockSpec((1, 4 * h), lambda i: (0, 0)),
        ],
        out_specs=pl.BlockSpec((rows, d), lambda i: (i, 0)),
        out_shape=jax.ShapeDtypeStruct((n, d), jnp.float32),
    )(x, acc, degp, wih_t, whh_t, b)


def kernel(x, edge_index, W_ih, W_hh, b_ih, b_hh):
    n, d = x.shape
    acc, degpacked = _sc_aggregate(x, edge_index[0], edge_index[1])
    deg = degpacked.reshape(NW, 10240)[:, :n].sum(axis=0)[:, None]
    b = (b_ih + b_hh).reshape(1, 2 * d)
    return _tc_lstm(x, W_ih.T, W_hh.T, b, acc, deg)


# speedup vs baseline: 11.6775x; 1.2016x over previous
import functools

import jax
import jax.numpy as jnp
from jax import lax
from jax.experimental import pallas as pl
from jax.experimental.pallas import tpu as pltpu
from jax.experimental.pallas import tpu_sc as plsc

NC = 2
NS = 16
NW = NC * NS

C = 80
RPT = 624


def _sc_aggregate(x, src_r, dst_r):
    n, d = x.shape
    k_chunks = src_r.shape[0] // (NW * C)
    extra = n - NS * RPT

    mesh = plsc.VectorSubcoreMesh(
        core_axis_name="c", subcore_axis_name="s",
        num_cores=NC, num_subcores=NS)

    @functools.partial(
        pl.kernel,
        out_type=(
            jax.ShapeDtypeStruct((NC, n, d), jnp.float32),
            jax.ShapeDtypeStruct((NW, 1, 10240), jnp.float32),
        ),
        mesh=mesh,
        compiler_params=pltpu.CompilerParams(needs_layout_passes=False),
        scratch_types=[
            pltpu.VMEM((C,), jnp.int32),
            pltpu.VMEM((C,), jnp.int32),
            pltpu.VMEM((C,), jnp.int32),
            pltpu.VMEM((C,), jnp.int32),
            pltpu.VMEM((C, d), jnp.float32),
            pltpu.VMEM((C, d), jnp.float32),
            pltpu.VMEM((10240,), jnp.float32),
            pltpu.VMEM_SHARED((n, d), jnp.float32),
            pltpu.SemaphoreType.DMA,
            pltpu.SemaphoreType.DMA,
            pltpu.SemaphoreType.DMA,
            pltpu.SemaphoreType.DMA,
        ],
    )
    def agg(x_hbm, src_hbm, dst_hbm, out_feat, out_deg,
            sidx0, didx0, sidx1, didx1, rows0, rows1,
            degh, acc_sh, sem, sem1, sem2, sem3):
        c = lax.axis_index("c")
        s = lax.axis_index("s")
        wid = s * NC + c

        zero16 = jnp.zeros((16,), jnp.float32)
        one16 = jnp.ones((16,), jnp.float32)
        iota16 = lax.iota(jnp.int32, 16)
        base = s * RPT
        ebase = wid * (k_chunks * C)

        def load_idx(kk, si, di):
            off = ebase + kk * C
            pltpu.async_copy(src_hbm.at[pl.ds(off, C)], si, sem2)
            pltpu.async_copy(dst_hbm.at[pl.ds(off, C)], di, sem3)
            pltpu.make_async_copy(src_hbm.at[pl.ds(off, C)], si, sem2).wait()
            pltpu.make_async_copy(dst_hbm.at[pl.ds(off, C)], di, sem3).wait()

        def hist(di):
            for j in range(C // 16):
                v = di[pl.ds(j * 16, 16)]
                plsc.addupdate_scatter(degh, [v], one16)

        def fill_batch_idx(si, q):
            for j in range(C // 16):
                si[pl.ds(j * 16, 16)] = base + q * C + j * 16 + iota16

        def zero_degh(r, carry):
            degh[pl.ds(r * 16, 16)] = zero16
            return carry

        lax.fori_loop(0, 640, zero_degh, 0)

        def zero_rows0(r, carry):
            for j in range(d // 16):
                rows0[r, pl.ds(j * 16, 16)] = zero16
            return carry

        lax.fori_loop(0, C, zero_rows0, 0)
        for q in range(8):
            fill_batch_idx(sidx0, q)
            pltpu.sync_copy(rows0, acc_sh.at[sidx0])
        plsc.subcore_barrier()

        load_idx(0, sidx0, didx0)
        pltpu.async_copy(x_hbm.at[sidx0], rows0, sem)
        hist(didx0)

        def feat_body(p, carry):
            kb = 2 * p + 1
            load_idx(kb, sidx1, didx1)
            pltpu.async_copy(x_hbm.at[sidx1], rows1, sem1)
            hist(didx1)
            pltpu.make_async_copy(x_hbm.at[sidx0], rows0, sem).wait()
            pltpu.sync_copy(rows0, acc_sh.at[didx0], add=True)
            load_idx(kb + 1, sidx0, didx0)
            pltpu.async_copy(x_hbm.at[sidx0], rows0, sem)
            hist(didx0)
            pltpu.make_async_copy(x_hbm.at[sidx1], rows1, sem1).wait()
            pltpu.sync_copy(rows1, acc_sh.at[didx1], add=True)
            return carry

        lax.fori_loop(0, (k_chunks - 1) // 2, feat_body, 0)
        pltpu.make_async_copy(x_hbm.at[sidx0], rows0, sem).wait()
        pltpu.sync_copy(rows0, acc_sh.at[didx0], add=True)
        pltpu.sync_copy(degh, out_deg.at[wid, 0])
        plsc.subcore_barrier()

        fill_batch_idx(sidx0, 0)
        pltpu.async_copy(acc_sh.at[sidx0], rows0, sem)
        for q in range(8):
            buf, sidx, sm = ((rows0, sidx0, sem) if q % 2 == 0
                             else (rows1, sidx1, sem1))
            if q < 7:
                nbuf, nsidx, nsm = ((rows0, sidx0, sem) if q % 2 == 1
                                    else (rows1, sidx1, sem1))
                fill_batch_idx(nsidx, q + 1)
                pltpu.async_copy(acc_sh.at[nsidx], nbuf, nsm)
            pltpu.make_async_copy(acc_sh.at[sidx], buf, sm).wait()
            pltpu.sync_copy(buf, out_feat.at[c, pl.ds(base + q * C, C), :])

    return agg(x, src_r, dst_r)


def _tc_lstm(x, wih_t, whh_t, b, acc, degp):
    n, d = x.shape
    h = d // 2
    rows = 1000
    grid = n // rows

    def body(x_ref, acc_ref, deg_ref, wih_ref, whh_ref, b_ref, out_ref):
        ft = (acc_ref[0] + acc_ref[1]) / jnp.maximum(deg_ref[...], 1.0)
        g_t = ft[:, :h]
        r_t = ft[:, h:]
        gates = jnp.dot(x_ref[...], wih_ref[...],
                        preferred_element_type=jnp.float32)
        gates += jnp.dot(g_t, whh_ref[...],
                         preferred_element_type=jnp.float32)
        gates += b_ref[...]
        i_g = jax.nn.sigmoid(gates[:, :h])
        f_g = jax.nn.sigmoid(gates[:, h:2 * h])
        g_g = jnp.tanh(gates[:, 2 * h:3 * h])
        o_g = jax.nn.sigmoid(gates[:, 3 * h:])
        c2 = f_g * r_t + i_g * g_g
        h2 = o_g * jnp.tanh(c2)
        out_ref[...] = jnp.concatenate([h2, c2], axis=1)

    return pl.pallas_call(
        body,
        grid=(grid,),
        in_specs=[
            pl.BlockSpec((rows, d), lambda i: (i, 0)),
            pl.BlockSpec((NC, rows, d), lambda i: (0, i, 0)),
            pl.BlockSpec((rows, 1), lambda i: (i, 0)),
            pl.BlockSpec((d, 4 * h), lambda i: (0, 0)),
            pl.BlockSpec((h, 4 * h), lambda i: (0, 0)),
            pl.BlockSpec((1, 4 * h), lambda i: (0, 0)),
        ],
        out_specs=pl.BlockSpec((rows, d), lambda i: (i, 0)),
        out_shape=jax.ShapeDtypeStruct((n, d), jnp.float32),
    )(x, acc, degp, wih_t, whh_t, b)


def kernel(x, edge_index, W_ih, W_hh, b_ih, b_hh):
    n, d = x.shape
    acc, degpacked = _sc_aggregate(x, edge_index[0], edge_index[1])
    deg = degpacked.reshape(NW, 10240)[:, :n].sum(axis=0)[:, None]
    b = (b_ih + b_hh).reshape(1, 2 * d)
    return _tc_lstm(x, W_ih.T, W_hh.T, b, acc, deg)
